# baseline (speedup 1.0000x reference)
import functools

import jax
import jax.numpy as jnp
from jax import lax
from jax.experimental import pallas as pl
from jax.experimental.pallas import tpu as pltpu
from jax.experimental.pallas import tpu_sc as plsc

NC = 2
NS = 16
NW = NC * NS
CHUNK = 128
NBUF = 8
FIRE = 4


def _sc_mesh():
  return plsc.VectorSubcoreMesh(
      core_axis_name="c", subcore_axis_name="s", num_cores=NC,
      num_subcores=NS)


def _deg_partials(dst3, acc_rows):
  k_chunks = dst3.shape[1]
  zrows = acc_rows // NS
  zpad = ((zrows + 15) // 16) * 16

  @functools.partial(
      pl.kernel,
      out_type=jax.ShapeDtypeStruct((NC * acc_rows,), jnp.float32),
      mesh=_sc_mesh(),
      compiler_params=pltpu.CompilerParams(use_tc_tiling_on_sc=False),
      scratch_types=[
          pltpu.VMEM((k_chunks, CHUNK), jnp.int32),
          pltpu.VMEM((CHUNK,), jnp.float32),
          pltpu.VMEM((zpad,), jnp.float32),
          pltpu.VMEM_SHARED((acc_rows,), jnp.float32),
      ],
  )
  def run(dst_hbm, out_hbm, dst_v, ones_v, z_v, acc_s):
    c = lax.axis_index("c")
    s = lax.axis_index("s")
    w = c * NS + s
    pltpu.sync_copy(dst_hbm.at[w], dst_v)

    @pl.loop(0, CHUNK // 16)
    def _fill(i):
      ones_v[pl.ds(i * 16, 16)] = jnp.ones((16,), jnp.float32)

    @pl.loop(0, zpad // 16)
    def _fillz(i):
      z_v[pl.ds(i * 16, 16)] = jnp.zeros((16,), jnp.float32)

    pltpu.sync_copy(z_v.at[pl.ds(0, zrows)], acc_s.at[pl.ds(s * zrows, zrows)])
    plsc.subcore_barrier()

    @pl.loop(0, k_chunks)
    def _edges(k):
      pltpu.sync_copy(ones_v, acc_s.at[dst_v.at[k]], add=True)

    plsc.subcore_barrier()
    pltpu.sync_copy(acc_s.at[pl.ds(s * zrows, zrows)], z_v.at[pl.ds(0, zrows)])
    pltpu.sync_copy(z_v.at[pl.ds(0, zrows)],
                    out_hbm.at[pl.ds(c * acc_rows + s * zrows, zrows)])

  return run(dst3)


def _prop_partials(gtab, src3, dst3, acc_rows, feat):
  k_chunks = src3.shape[1]
  zrows = acc_rows // NS
  offs = list(range(0, feat - 15, 16))
  if feat % 16:
    offs.append(feat - 16)

  @functools.partial(
      pl.kernel,
      out_type=jax.ShapeDtypeStruct((NC, acc_rows, feat), jnp.float32),
      mesh=_sc_mesh(),
      compiler_params=pltpu.CompilerParams(use_tc_tiling_on_sc=False),
      scratch_types=[
          pltpu.VMEM((k_chunks, CHUNK), jnp.int32),
          pltpu.VMEM((k_chunks, CHUNK), jnp.int32),
          pltpu.VMEM((NBUF, CHUNK, feat), jnp.float32),
          pltpu.VMEM((zrows, feat), jnp.float32),
          pltpu.VMEM_SHARED((acc_rows, feat), jnp.float32),
      ] + [pltpu.SemaphoreType.DMA] * (2 * NBUF),
  )
  def run(g_hbm, src_hbm, dst_hbm, out_hbm,
          src_v, dst_v, rows_v, z_v, acc_s, *sems):
    gsem = sems[:NBUF]
    ssem = sems[NBUF:]
    c = lax.axis_index("c")
    s = lax.axis_index("s")
    w = c * NS + s
    pltpu.sync_copy(src_hbm.at[w], src_v)
    pltpu.sync_copy(dst_hbm.at[w], dst_v)

    @pl.loop(0, zrows)
    def _fillz(r):
      for o in offs:
        z_v[r, pl.ds(o, 16)] = jnp.zeros((16,), jnp.float32)

    pltpu.sync_copy(z_v, acc_s.at[pl.ds(s * zrows, zrows)])
    plsc.subcore_barrier()

    def wait_gather(k, b):
      pltpu.make_async_copy(g_hbm.at[src_v.at[k]], rows_v.at[b],
                            gsem[b]).wait()

    def wait_scatter(k, b):
      pltpu.make_async_copy(rows_v.at[b], acc_s.at[dst_v.at[k]],
                            ssem[b]).wait()

    for b in range(FIRE):
      pltpu.async_copy(g_hbm.at[src_v.at[b]], rows_v.at[b], gsem[b])

    for b in range(NBUF):
      k = b
      wait_gather(k, b)
      pltpu.async_copy(rows_v.at[b], acc_s.at[dst_v.at[k]], ssem[b],
                       add=True)
      nb = (k + FIRE) % NBUF
      if k + FIRE < NBUF:
        pltpu.async_copy(g_hbm.at[src_v.at[k + FIRE]], rows_v.at[nb],
                         gsem[nb])
      else:
        wait_scatter(k + FIRE - NBUF, nb)
        pltpu.async_copy(g_hbm.at[src_v.at[k + FIRE]], rows_v.at[nb],
                         gsem[nb])

    @pl.loop(1, k_chunks // NBUF)
    def _edges(g):
      for b in range(NBUF):
        k = g * NBUF + b
        wait_gather(k, b)
        pltpu.async_copy(rows_v.at[b], acc_s.at[dst_v.at[k]], ssem[b],
                         add=True)
        nxt = k + FIRE
        nb = (b + FIRE) % NBUF

        @pl.when(nxt < k_chunks)
        def _refire():
          wait_scatter(nxt - NBUF, nb)
          pltpu.async_copy(g_hbm.at[src_v.at[nxt]], rows_v.at[nb], gsem[nb])

    for b in range(NBUF):
      k = k_chunks - NBUF + b
      wait_scatter(k, k % NBUF)

    plsc.subcore_barrier()
    pltpu.sync_copy(acc_s.at[pl.ds(s * zrows, zrows)], z_v)
    pltpu.sync_copy(z_v, out_hbm.at[c, pl.ds(s * zrows, zrows)])

  return run(gtab, src3, dst3)


def _tc_first(deg_p, x, w1, bm):
  n, d_in = x.shape
  hid = w1.shape[1]
  grid = n // bm

  def body(deg_ref, x_ref, w1_ref, dis_ref, g1_ref):
    deg = deg_ref[0] + deg_ref[1] + 1.0
    dis = lax.rsqrt(deg)
    h1 = jnp.dot(x_ref[...], w1_ref[...], preferred_element_type=jnp.float32)
    dis_ref[...] = dis
    g1_ref[...] = dis * h1

  return pl.pallas_call(
      body,
      grid=(grid,),
      in_specs=[
          pl.BlockSpec((NC, bm, 1), lambda i: (0, i, 0)),
          pl.BlockSpec((bm, d_in), lambda i: (i, 0)),
          pl.BlockSpec((d_in, hid), lambda i: (0, 0)),
      ],
      out_specs=[
          pl.BlockSpec((bm, 1), lambda i: (i, 0)),
          pl.BlockSpec((bm, hid), lambda i: (i, 0)),
      ],
      out_shape=[
          jax.ShapeDtypeStruct((n, 1), jnp.float32),
          jax.ShapeDtypeStruct((n, hid), jnp.float32),
      ],
  )(deg_p, x, w1)


def _tc_mid(acc_p, g1, dis, b1, w2, bm):
  n, hid = g1.shape
  ncls = w2.shape[1]
  grid = n // bm

  def body(acc_ref, g1_ref, dis_ref, b1_ref, w2_ref, g2_ref):
    a = acc_ref[0] + acc_ref[1] + g1_ref[...]
    z = jnp.maximum(dis_ref[...] * a + b1_ref[...], 0.0)
    h2 = jnp.dot(z, w2_ref[...], preferred_element_type=jnp.float32)
    g2_ref[...] = dis_ref[...] * h2

  return pl.pallas_call(
      body,
      grid=(grid,),
      in_specs=[
          pl.BlockSpec((NC, bm, hid), lambda i: (0, i, 0)),
          pl.BlockSpec((bm, hid), lambda i: (i, 0)),
          pl.BlockSpec((bm, 1), lambda i: (i, 0)),
          pl.BlockSpec((1, hid), lambda i: (0, 0)),
          pl.BlockSpec((hid, ncls), lambda i: (0, 0)),
      ],
      out_specs=pl.BlockSpec((bm, ncls), lambda i: (i, 0)),
      out_shape=jax.ShapeDtypeStruct((n, ncls), jnp.float32),
  )(acc_p, g1, dis, b1, w2)


def _tc_last(acc_p, g2, dis, b2, bm):
  n, ncls = g2.shape
  grid = n // bm

  def body(acc_ref, g2_ref, dis_ref, b2_ref, out_ref):
    a = acc_ref[0] + acc_ref[1] + g2_ref[...]
    out_ref[...] = dis_ref[...] * a + b2_ref[...]

  return pl.pallas_call(
      body,
      grid=(grid,),
      in_specs=[
          pl.BlockSpec((NC, bm, ncls), lambda i: (0, i, 0)),
          pl.BlockSpec((bm, ncls), lambda i: (i, 0)),
          pl.BlockSpec((bm, 1), lambda i: (i, 0)),
          pl.BlockSpec((1, ncls), lambda i: (0, 0)),
      ],
      out_specs=pl.BlockSpec((bm, ncls), lambda i: (i, 0)),
      out_shape=jax.ShapeDtypeStruct((n, ncls), jnp.float32),
  )(acc_p, g2, dis, b2)


def kernel(x, edge_index, W1, b1, W2, b2):
  n, _ = x.shape
  hid = W1.shape[1]
  ncls = W2.shape[1]
  e = edge_index.shape[1]

  acc_rows = ((n + 1 + 127) // 128) * 128
  zrows = acc_rows // NS
  bm = 2000

  ew = NW * CHUNK
  k_chunks = ((e + ew - 1) // ew + NBUF - 1) // NBUF * NBUF
  e_pad = k_chunks * ew
  src = edge_index[0]
  dst = edge_index[1]
  pad = e_pad - e
  src3 = jnp.concatenate(
      [src, jnp.zeros((pad,), jnp.int32)]).reshape(NW, k_chunks, CHUNK)
  dst3 = jnp.concatenate(
      [dst, jnp.full((pad,), n, jnp.int32)]).reshape(NW, k_chunks, CHUNK)

  deg_p = _deg_partials(dst3, acc_rows)
  dis, g1 = _tc_first(deg_p.reshape(NC, acc_rows, 1), x, W1, bm)
  acc1 = _prop_partials(g1, src3, dst3, acc_rows, hid)
  g2 = _tc_mid(acc1, g1, dis, b1.reshape(1, hid), W2, bm)
  acc2 = _prop_partials(g2, src3, dst3, acc_rows, ncls)
  return _tc_last(acc2, g2, dis, b2.reshape(1, ncls), bm)


# speedup vs baseline: 1.4937x; 1.4937x over previous
import functools

import jax
import jax.numpy as jnp
from jax import lax
from jax.experimental import pallas as pl
from jax.experimental.pallas import tpu as pltpu
from jax.experimental.pallas import tpu_sc as plsc

NC = 2
NS = 16
NW = NC * NS
CHUNK = 128
NBUF = 8
FIRE = 4


def _sc_mesh():
  return plsc.VectorSubcoreMesh(
      core_axis_name="c", subcore_axis_name="s", num_cores=NC,
      num_subcores=NS)


def _deg_partials(dst3, acc_rows):
  k_chunks = dst3.shape[1]
  zrows = acc_rows // NS
  zpad = ((zrows + 15) // 16) * 16

  @functools.partial(
      pl.kernel,
      out_type=jax.ShapeDtypeStruct((NC * acc_rows,), jnp.float32),
      mesh=_sc_mesh(),
      compiler_params=pltpu.CompilerParams(use_tc_tiling_on_sc=False),
      scratch_types=[
          pltpu.VMEM((k_chunks, CHUNK), jnp.int32),
          pltpu.VMEM((CHUNK,), jnp.float32),
          pltpu.VMEM((zpad,), jnp.float32),
          pltpu.VMEM_SHARED((acc_rows,), jnp.float32),
      ],
  )
  def run(dst_hbm, out_hbm, dst_v, ones_v, z_v, acc_s):
    c = lax.axis_index("c")
    s = lax.axis_index("s")
    w = c * NS + s
    pltpu.sync_copy(dst_hbm.at[w], dst_v)

    @pl.loop(0, CHUNK // 16)
    def _fill(i):
      ones_v[pl.ds(i * 16, 16)] = jnp.ones((16,), jnp.float32)

    @pl.loop(0, zpad // 16)
    def _fillz(i):
      z_v[pl.ds(i * 16, 16)] = jnp.zeros((16,), jnp.float32)

    pltpu.sync_copy(z_v.at[pl.ds(0, zrows)], acc_s.at[pl.ds(s * zrows, zrows)])
    plsc.subcore_barrier()

    @pl.loop(0, k_chunks)
    def _edges(k):
      pltpu.sync_copy(ones_v, acc_s.at[dst_v.at[k]], add=True)

    plsc.subcore_barrier()
    pltpu.sync_copy(acc_s.at[pl.ds(s * zrows, zrows)], z_v.at[pl.ds(0, zrows)])
    pltpu.sync_copy(z_v.at[pl.ds(0, zrows)],
                    out_hbm.at[pl.ds(c * acc_rows + s * zrows, zrows)])

  return run(dst3)


def _prop_partials(gtab, src3, dst3, acc_rows, feat, stage):
  k_chunks = src3.shape[1]
  zrows = acc_rows // NS
  offs = list(range(0, feat - 15, 16))
  if feat % 16:
    offs.append(feat - 16)

  @functools.partial(
      pl.kernel,
      out_type=jax.ShapeDtypeStruct((NC, acc_rows, feat), jnp.float32),
      mesh=_sc_mesh(),
      compiler_params=pltpu.CompilerParams(use_tc_tiling_on_sc=False),
      scratch_types=[
          pltpu.VMEM((k_chunks, CHUNK), jnp.int32),
          pltpu.VMEM((k_chunks, CHUNK), jnp.int32),
          pltpu.VMEM((NBUF * CHUNK, feat), jnp.float32),
          pltpu.VMEM_SHARED((acc_rows, feat), jnp.float32),
      ] + ([pltpu.VMEM_SHARED((acc_rows, feat), jnp.float32)] if stage
           else [])
      + [pltpu.SemaphoreType.DMA] * (2 * NBUF),
  )
  def run(g_hbm, src_hbm, dst_hbm, out_hbm,
          src_v, dst_v, rows_v, acc_s, *rest):
    if stage:
      gtab_s, *sems = rest
    else:
      sems = rest
    gsem = sems[:NBUF]
    ssem = sems[NBUF:]
    gsrc = gtab_s if stage else g_hbm

    def buf(b):
      return rows_v.at[pl.ds(b * CHUNK, CHUNK)]

    z_v = rows_v.at[pl.ds(0, zrows)]
    c = lax.axis_index("c")
    s = lax.axis_index("s")
    w = c * NS + s
    n_rows = g_hbm.shape[0]
    last = n_rows - (NS - 1) * zrows
    pltpu.sync_copy(src_hbm.at[w], src_v)
    pltpu.sync_copy(dst_hbm.at[w], dst_v)

    if stage:
      @pl.when(s < NS - 1)
      def _stage_full():
        pltpu.sync_copy(g_hbm.at[pl.ds(s * zrows, zrows)], z_v)
        pltpu.sync_copy(z_v, gtab_s.at[pl.ds(s * zrows, zrows)])

      @pl.when(s == NS - 1)
      def _stage_last():
        pltpu.sync_copy(g_hbm.at[pl.ds((NS - 1) * zrows, last)],
                        rows_v.at[pl.ds(0, last)])
        pltpu.sync_copy(rows_v.at[pl.ds(0, last)],
                        gtab_s.at[pl.ds((NS - 1) * zrows, last)])

    @pl.loop(0, zrows)
    def _fillz(r):
      for o in offs:
        rows_v[r, pl.ds(o, 16)] = jnp.zeros((16,), jnp.float32)

    pltpu.sync_copy(z_v, acc_s.at[pl.ds(s * zrows, zrows)])
    plsc.subcore_barrier()

    def wait_gather(k, b):
      pltpu.make_async_copy(gsrc.at[src_v.at[k]], buf(b),
                            gsem[b]).wait()

    def wait_scatter(k, b):
      pltpu.make_async_copy(buf(b), acc_s.at[dst_v.at[k]],
                            ssem[b]).wait()

    for b in range(FIRE):
      pltpu.async_copy(gsrc.at[src_v.at[b]], buf(b), gsem[b])

    for b in range(NBUF):
      k = b
      wait_gather(k, b)
      pltpu.async_copy(buf(b), acc_s.at[dst_v.at[k]], ssem[b],
                       add=True)
      nb = (k + FIRE) % NBUF
      if k + FIRE < NBUF:
        pltpu.async_copy(gsrc.at[src_v.at[k + FIRE]], buf(nb),
                         gsem[nb])
      else:
        wait_scatter(k + FIRE - NBUF, nb)
        pltpu.async_copy(gsrc.at[src_v.at[k + FIRE]], buf(nb),
                         gsem[nb])

    @pl.loop(1, k_chunks // NBUF)
    def _edges(g):
      for b in range(NBUF):
        k = g * NBUF + b
        wait_gather(k, b)
        pltpu.async_copy(buf(b), acc_s.at[dst_v.at[k]], ssem[b],
                         add=True)
        nxt = k + FIRE
        nb = (b + FIRE) % NBUF

        @pl.when(nxt < k_chunks)
        def _refire():
          wait_scatter(nxt - NBUF, nb)
          pltpu.async_copy(gsrc.at[src_v.at[nxt]], buf(nb), gsem[nb])

    for b in range(NBUF):
      k = k_chunks - NBUF + b
      wait_scatter(k, k % NBUF)

    plsc.subcore_barrier()
    pltpu.sync_copy(acc_s.at[pl.ds(s * zrows, zrows)], z_v)
    pltpu.sync_copy(z_v, out_hbm.at[c, pl.ds(s * zrows, zrows)])

  return run(gtab, src3, dst3)


def _tc_first(deg_p, x, w1, bm):
  n, d_in = x.shape
  hid = w1.shape[1]
  grid = n // bm

  def body(deg_ref, x_ref, w1_ref, dis_ref, g1_ref):
    deg = deg_ref[0] + deg_ref[1] + 1.0
    dis = lax.rsqrt(deg)
    h1 = jnp.dot(x_ref[...], w1_ref[...], preferred_element_type=jnp.float32)
    dis_ref[...] = dis
    g1_ref[...] = dis * h1

  return pl.pallas_call(
      body,
      grid=(grid,),
      in_specs=[
          pl.BlockSpec((NC, bm, 1), lambda i: (0, i, 0)),
          pl.BlockSpec((bm, d_in), lambda i: (i, 0)),
          pl.BlockSpec((d_in, hid), lambda i: (0, 0)),
      ],
      out_specs=[
          pl.BlockSpec((bm, 1), lambda i: (i, 0)),
          pl.BlockSpec((bm, hid), lambda i: (i, 0)),
      ],
      out_shape=[
          jax.ShapeDtypeStruct((n, 1), jnp.float32),
          jax.ShapeDtypeStruct((n, hid), jnp.float32),
      ],
  )(deg_p, x, w1)


def _tc_mid(acc_p, g1, dis, b1, w2, bm):
  n, hid = g1.shape
  ncls = w2.shape[1]
  grid = n // bm

  def body(acc_ref, g1_ref, dis_ref, b1_ref, w2_ref, g2_ref):
    a = acc_ref[0] + acc_ref[1] + g1_ref[...]
    z = jnp.maximum(dis_ref[...] * a + b1_ref[...], 0.0)
    h2 = jnp.dot(z, w2_ref[...], preferred_element_type=jnp.float32)
    g2_ref[...] = dis_ref[...] * h2

  return pl.pallas_call(
      body,
      grid=(grid,),
      in_specs=[
          pl.BlockSpec((NC, bm, hid), lambda i: (0, i, 0)),
          pl.BlockSpec((bm, hid), lambda i: (i, 0)),
          pl.BlockSpec((bm, 1), lambda i: (i, 0)),
          pl.BlockSpec((1, hid), lambda i: (0, 0)),
          pl.BlockSpec((hid, ncls), lambda i: (0, 0)),
      ],
      out_specs=pl.BlockSpec((bm, ncls), lambda i: (i, 0)),
      out_shape=jax.ShapeDtypeStruct((n, ncls), jnp.float32),
  )(acc_p, g1, dis, b1, w2)


def _tc_last(acc_p, g2, dis, b2, bm):
  n, ncls = g2.shape
  grid = n // bm

  def body(acc_ref, g2_ref, dis_ref, b2_ref, out_ref):
    a = acc_ref[0] + acc_ref[1] + g2_ref[...]
    out_ref[...] = dis_ref[...] * a + b2_ref[...]

  return pl.pallas_call(
      body,
      grid=(grid,),
      in_specs=[
          pl.BlockSpec((NC, bm, ncls), lambda i: (0, i, 0)),
          pl.BlockSpec((bm, ncls), lambda i: (i, 0)),
          pl.BlockSpec((bm, 1), lambda i: (i, 0)),
          pl.BlockSpec((1, ncls), lambda i: (0, 0)),
      ],
      out_specs=pl.BlockSpec((bm, ncls), lambda i: (i, 0)),
      out_shape=jax.ShapeDtypeStruct((n, ncls), jnp.float32),
  )(acc_p, g2, dis, b2)


def kernel(x, edge_index, W1, b1, W2, b2):
  n, _ = x.shape
  hid = W1.shape[1]
  ncls = W2.shape[1]
  e = edge_index.shape[1]

  acc_rows = ((n + 1 + 127) // 128) * 128
  zrows = acc_rows // NS
  bm = 2000

  ew = NW * CHUNK
  k_chunks = ((e + ew - 1) // ew + NBUF - 1) // NBUF * NBUF
  e_pad = k_chunks * ew
  src = edge_index[0]
  dst = edge_index[1]
  pad = e_pad - e
  src3 = jnp.concatenate(
      [src, jnp.zeros((pad,), jnp.int32)]).reshape(NW, k_chunks, CHUNK)
  dst3 = jnp.concatenate(
      [dst, jnp.full((pad,), n, jnp.int32)]).reshape(NW, k_chunks, CHUNK)

  deg_p = _deg_partials(dst3, acc_rows)
  dis, g1 = _tc_first(deg_p.reshape(NC, acc_rows, 1), x, W1, bm)
  acc1 = _prop_partials(g1, src3, dst3, acc_rows, hid, stage=False)
  g2 = _tc_mid(acc1, g1, dis, b1.reshape(1, hid), W2, bm)
  acc2 = _prop_partials(g2, src3, dst3, acc_rows, ncls, stage=True)
  return _tc_last(acc2, g2, dis, b2.reshape(1, ncls), bm)


# speedup vs baseline: 1.7273x; 1.1564x over previous
import functools

import jax
import jax.numpy as jnp
from jax import lax
from jax.experimental import pallas as pl
from jax.experimental.pallas import tpu as pltpu
from jax.experimental.pallas import tpu_sc as plsc

NC = 2
NS = 16
NW = NC * NS
CHUNK = 128
NBUF = 8
FIRE = 4


def _sc_mesh():
  return plsc.VectorSubcoreMesh(
      core_axis_name="c", subcore_axis_name="s", num_cores=NC,
      num_subcores=NS)


def _deg_partials(dst3, acc_rows):
  k_chunks = dst3.shape[1]
  zrows = acc_rows // NS
  zpad = ((zrows + 15) // 16) * 16

  @functools.partial(
      pl.kernel,
      out_type=jax.ShapeDtypeStruct((NC * acc_rows,), jnp.float32),
      mesh=_sc_mesh(),
      compiler_params=pltpu.CompilerParams(use_tc_tiling_on_sc=False),
      scratch_types=[
          pltpu.VMEM((k_chunks, CHUNK), jnp.int32),
          pltpu.VMEM((CHUNK,), jnp.float32),
          pltpu.VMEM((zpad,), jnp.float32),
          pltpu.VMEM_SHARED((acc_rows,), jnp.float32),
      ],
  )
  def run(dst_hbm, out_hbm, dst_v, ones_v, z_v, acc_s):
    c = lax.axis_index("c")
    s = lax.axis_index("s")
    w = c * NS + s
    pltpu.sync_copy(dst_hbm.at[w], dst_v)

    @pl.loop(0, CHUNK // 16)
    def _fill(i):
      ones_v[pl.ds(i * 16, 16)] = jnp.ones((16,), jnp.float32)

    @pl.loop(0, zpad // 16)
    def _fillz(i):
      z_v[pl.ds(i * 16, 16)] = jnp.zeros((16,), jnp.float32)

    pltpu.sync_copy(z_v.at[pl.ds(0, zrows)], acc_s.at[pl.ds(s * zrows, zrows)])
    plsc.subcore_barrier()

    @pl.loop(0, k_chunks)
    def _edges(k):
      pltpu.sync_copy(ones_v, acc_s.at[dst_v.at[k]], add=True)

    plsc.subcore_barrier()
    pltpu.sync_copy(acc_s.at[pl.ds(s * zrows, zrows)], z_v.at[pl.ds(0, zrows)])
    pltpu.sync_copy(z_v.at[pl.ds(0, zrows)],
                    out_hbm.at[pl.ds(c * acc_rows + s * zrows, zrows)])

  return run(dst3)


def _prop_partials(gtab, src3, dst3, acc_rows, feat, stage):
  k_chunks = src3.shape[1]
  zrows = acc_rows // NS
  offs = list(range(0, feat - 15, 16))
  if feat % 16:
    offs.append(feat - 16)

  @functools.partial(
      pl.kernel,
      out_type=jax.ShapeDtypeStruct((NC, acc_rows, feat), jnp.float32),
      mesh=_sc_mesh(),
      compiler_params=pltpu.CompilerParams(use_tc_tiling_on_sc=False),
      scratch_types=[
          pltpu.VMEM((k_chunks, CHUNK), jnp.int32),
          pltpu.VMEM((k_chunks, CHUNK), jnp.int32),
          pltpu.VMEM((NBUF * CHUNK, feat), jnp.float32),
          pltpu.VMEM_SHARED((acc_rows, feat), jnp.float32),
      ] + ([pltpu.VMEM_SHARED((acc_rows, feat), jnp.float32)] if stage
           else [])
      + [pltpu.SemaphoreType.DMA] * (2 * NBUF),
  )
  def run(g_hbm, src_hbm, dst_hbm, out_hbm,
          src_v, dst_v, rows_v, acc_s, *rest):
    if stage:
      gtab_s, *sems = rest
    else:
      sems = rest
    gsem = sems[:NBUF]
    ssem = sems[NBUF:]
    gsrc = gtab_s if stage else g_hbm

    def buf(b):
      return rows_v.at[pl.ds(b * CHUNK, CHUNK)]

    z_v = rows_v.at[pl.ds(0, zrows)]
    c = lax.axis_index("c")
    s = lax.axis_index("s")
    w = c * NS + s
    n_rows = g_hbm.shape[0]
    last = n_rows - (NS - 1) * zrows
    pltpu.sync_copy(src_hbm.at[w], src_v)
    pltpu.sync_copy(dst_hbm.at[w], dst_v)

    if stage:
      @pl.when(s < NS - 1)
      def _stage_full():
        pltpu.sync_copy(g_hbm.at[pl.ds(s * zrows, zrows)], z_v)
        pltpu.sync_copy(z_v, gtab_s.at[pl.ds(s * zrows, zrows)])

      @pl.when(s == NS - 1)
      def _stage_last():
        pltpu.sync_copy(g_hbm.at[pl.ds((NS - 1) * zrows, last)],
                        rows_v.at[pl.ds(0, last)])
        pltpu.sync_copy(rows_v.at[pl.ds(0, last)],
                        gtab_s.at[pl.ds((NS - 1) * zrows, last)])

    @pl.loop(0, zrows)
    def _fillz(r):
      for o in offs:
        rows_v[r, pl.ds(o, 16)] = jnp.zeros((16,), jnp.float32)

    pltpu.sync_copy(z_v, acc_s.at[pl.ds(s * zrows, zrows)])
    plsc.subcore_barrier()

    def wait_gather(k, b):
      pltpu.make_async_copy(gsrc.at[src_v.at[k]], buf(b),
                            gsem[b]).wait()

    def wait_scatter(k, b):
      pltpu.make_async_copy(buf(b), acc_s.at[dst_v.at[k]],
                            ssem[b]).wait()

    for b in range(FIRE):
      pltpu.async_copy(gsrc.at[src_v.at[b]], buf(b), gsem[b])

    for b in range(NBUF):
      k = b
      wait_gather(k, b)
      pltpu.async_copy(buf(b), acc_s.at[dst_v.at[k]], ssem[b],
                       add=True)
      nb = (k + FIRE) % NBUF
      if k + FIRE < NBUF:
        pltpu.async_copy(gsrc.at[src_v.at[k + FIRE]], buf(nb),
                         gsem[nb])
      else:
        wait_scatter(k + FIRE - NBUF, nb)
        pltpu.async_copy(gsrc.at[src_v.at[k + FIRE]], buf(nb),
                         gsem[nb])

    @pl.loop(1, k_chunks // NBUF)
    def _edges(g):
      for b in range(NBUF):
        k = g * NBUF + b
        wait_gather(k, b)
        pltpu.async_copy(buf(b), acc_s.at[dst_v.at[k]], ssem[b],
                         add=True)
        nxt = k + FIRE
        nb = (b + FIRE) % NBUF

        @pl.when(nxt < k_chunks)
        def _refire():
          wait_scatter(nxt - NBUF, nb)
          pltpu.async_copy(gsrc.at[src_v.at[nxt]], buf(nb), gsem[nb])

    for b in range(NBUF):
      k = k_chunks - NBUF + b
      wait_scatter(k, k % NBUF)

    plsc.subcore_barrier()
    pltpu.sync_copy(acc_s.at[pl.ds(s * zrows, zrows)], z_v)
    pltpu.sync_copy(z_v, out_hbm.at[c, pl.ds(s * zrows, zrows)])

  return run(gtab, src3, dst3)


def _tc_first(deg_p, x, w1, bm):
  n, d_in = x.shape
  hid = w1.shape[1]
  grid = n // bm

  def body(deg_ref, x_ref, w1_ref, dis_ref, g1_ref):
    deg = deg_ref[0] + deg_ref[1] + 1.0
    dis = lax.rsqrt(deg)
    h1 = jnp.dot(x_ref[...], w1_ref[...], preferred_element_type=jnp.float32)
    dis_ref[...] = dis
    g1_ref[...] = dis * h1

  return pl.pallas_call(
      body,
      grid=(grid,),
      in_specs=[
          pl.BlockSpec((NC, bm, 1), lambda i: (0, i, 0)),
          pl.BlockSpec((bm, d_in), lambda i: (i, 0)),
          pl.BlockSpec((d_in, hid), lambda i: (0, 0)),
      ],
      out_specs=[
          pl.BlockSpec((bm, 1), lambda i: (i, 0)),
          pl.BlockSpec((bm, hid), lambda i: (i, 0)),
      ],
      out_shape=[
          jax.ShapeDtypeStruct((n, 1), jnp.float32),
          jax.ShapeDtypeStruct((n, hid), jnp.float32),
      ],
  )(deg_p, x, w1)


def _tc_mid(acc_p, g1, dis, b1, w2, bm):
  n, hid = g1.shape
  ncls = w2.shape[1]
  grid = n // bm

  def body(acc_ref, g1_ref, dis_ref, b1_ref, w2_ref, g2_ref):
    a = acc_ref[0] + acc_ref[1] + g1_ref[...]
    z = jnp.maximum(dis_ref[...] * a + b1_ref[...], 0.0)
    h2 = jnp.dot(z, w2_ref[...], preferred_element_type=jnp.float32)
    g2_ref[...] = dis_ref[...] * h2

  return pl.pallas_call(
      body,
      grid=(grid,),
      in_specs=[
          pl.BlockSpec((NC, bm, hid), lambda i: (0, i, 0)),
          pl.BlockSpec((bm, hid), lambda i: (i, 0)),
          pl.BlockSpec((bm, 1), lambda i: (i, 0)),
          pl.BlockSpec((1, hid), lambda i: (0, 0)),
          pl.BlockSpec((hid, ncls), lambda i: (0, 0)),
      ],
      out_specs=pl.BlockSpec((bm, ncls), lambda i: (i, 0)),
      out_shape=jax.ShapeDtypeStruct((n, ncls), jnp.float32),
  )(acc_p, g1, dis, b1, w2)


def _tc_last(acc_p, g2, dis, b2, bm):
  n, ncls = g2.shape
  grid = n // bm

  def body(acc_ref, g2_ref, dis_ref, b2_ref, out_ref):
    a = acc_ref[0] + acc_ref[1] + g2_ref[...]
    out_ref[...] = dis_ref[...] * a + b2_ref[...]

  return pl.pallas_call(
      body,
      grid=(grid,),
      in_specs=[
          pl.BlockSpec((NC, bm, ncls), lambda i: (0, i, 0)),
          pl.BlockSpec((bm, ncls), lambda i: (i, 0)),
          pl.BlockSpec((bm, 1), lambda i: (i, 0)),
          pl.BlockSpec((1, ncls), lambda i: (0, 0)),
      ],
      out_specs=pl.BlockSpec((bm, ncls), lambda i: (i, 0)),
      out_shape=jax.ShapeDtypeStruct((n, ncls), jnp.float32),
  )(acc_p, g2, dis, b2)


def kernel(x, edge_index, W1, b1, W2, b2):
  n, _ = x.shape
  hid = W1.shape[1]
  ncls = W2.shape[1]
  e = edge_index.shape[1]

  acc_rows = ((n + 1 + 127) // 128) * 128
  zrows = acc_rows // NS
  bm = 2000

  ew = NW * CHUNK
  k_chunks = ((e + ew - 1) // ew + NBUF - 1) // NBUF * NBUF
  e_pad = k_chunks * ew
  src = edge_index[0]
  dst = edge_index[1]
  pad = e_pad - e
  src3 = jnp.concatenate(
      [src, jnp.zeros((pad,), jnp.int32)]).reshape(NW, k_chunks, CHUNK)
  dst3 = jnp.concatenate(
      [dst, jnp.full((pad,), n, jnp.int32)]).reshape(NW, k_chunks, CHUNK)

  deg_p = _deg_partials(dst3, acc_rows)
  dis, g1 = _tc_first(deg_p.reshape(NC, acc_rows, 1), x, W1, bm)
  acc1 = _prop_partials(g1, src3, dst3, acc_rows, hid, stage=True)
  g2 = _tc_mid(acc1, g1, dis, b1.reshape(1, hid), W2, bm)
  acc2 = _prop_partials(g2, src3, dst3, acc_rows, ncls, stage=True)
  return _tc_last(acc2, g2, dis, b2.reshape(1, ncls), bm)


# speedup vs baseline: 1.7815x; 1.0314x over previous
import functools

import jax
import jax.numpy as jnp
from jax import lax
from jax.experimental import pallas as pl
from jax.experimental.pallas import tpu as pltpu
from jax.experimental.pallas import tpu_sc as plsc

NC = 2
NS = 16
NW = NC * NS
CHUNK = 128
NBUF = 8
FIRE = 4
DSEM = 8


def _sc_mesh():
  return plsc.VectorSubcoreMesh(
      core_axis_name="c", subcore_axis_name="s", num_cores=NC,
      num_subcores=NS)


def _qrsqrt(d):
  i = plsc.bitcast(d, jnp.int32)
  i = jnp.int32(0x5F3759DF) - lax.shift_right_logical(i, 1)
  y = plsc.bitcast(i, jnp.float32)
  for _ in range(3):
    y = y * (1.5 - 0.5 * d * y * y)
  return y


def _row_offs(feat):
  offs = list(range(0, feat - 15, 16))
  if feat % 16:
    offs.append(feat - 16)
  return offs


def _fill_zero_rows(rows_v, nrows, offs):
  @pl.loop(0, nrows)
  def _fz(r):
    for o in offs:
      rows_v[r, pl.ds(o, 16)] = jnp.zeros((16,), jnp.float32)


def _edge_ring(gtab_s, acc_s, src_v, dst_v, rows_v, gsem, ssem, k_chunks):
  def buf(b):
    return rows_v.at[pl.ds(b * CHUNK, CHUNK)]

  def wait_gather(k, b):
    pltpu.make_async_copy(gtab_s.at[src_v.at[k]], buf(b), gsem[b]).wait()

  def wait_scatter(k, b):
    pltpu.make_async_copy(buf(b), acc_s.at[dst_v.at[k]], ssem[b]).wait()

  for b in range(FIRE):
    pltpu.async_copy(gtab_s.at[src_v.at[b]], buf(b), gsem[b])

  for b in range(NBUF):
    k = b
    wait_gather(k, b)
    pltpu.async_copy(buf(b), acc_s.at[dst_v.at[k]], ssem[b], add=True)
    nb = (k + FIRE) % NBUF
    if k + FIRE >= NBUF:
      wait_scatter(k + FIRE - NBUF, nb)
    pltpu.async_copy(gtab_s.at[src_v.at[k + FIRE]], buf(nb), gsem[nb])

  @pl.loop(1, k_chunks // NBUF)
  def _edges(g):
    for b in range(NBUF):
      k = g * NBUF + b
      wait_gather(k, b)
      pltpu.async_copy(buf(b), acc_s.at[dst_v.at[k]], ssem[b], add=True)
      nxt = k + FIRE
      nb = (b + FIRE) % NBUF

      @pl.when(nxt < k_chunks)
      def _refire():
        wait_scatter(nxt - NBUF, nb)
        pltpu.async_copy(gtab_s.at[src_v.at[nxt]], buf(nb), gsem[nb])

  for b in range(NBUF):
    k = k_chunks - NBUF + b
    wait_scatter(k, k % NBUF)


def _stage_scaled(h_hbm, dis_v, rows_v, gtab_s, s, zrows, feat, n_rows):
  last = n_rows - (NS - 1) * zrows
  offs = _row_offs(feat)

  def scale(nr):
    @pl.loop(0, (nr + 15) // 16)
    def _sc(i):
      dv = dis_v[pl.ds(i * 16, 16)]
      for j in range(16):
        r = i * 16 + j
        d = lax.broadcast_in_dim(dv[j], (16,), ())
        for o in offs:
          rows_v[r, pl.ds(o, 16)] = rows_v[r, pl.ds(o, 16)] * d

  @pl.when(s < NS - 1)
  def _full():
    pltpu.sync_copy(h_hbm.at[pl.ds(s * zrows, zrows)],
                    rows_v.at[pl.ds(0, zrows)])
    scale(zrows)
    pltpu.sync_copy(rows_v.at[pl.ds(0, zrows)],
                    gtab_s.at[pl.ds(s * zrows, zrows)])

  @pl.when(s == NS - 1)
  def _last():
    pltpu.sync_copy(h_hbm.at[pl.ds((NS - 1) * zrows, last)],
                    rows_v.at[pl.ds(0, last)])
    scale(last)
    pltpu.sync_copy(rows_v.at[pl.ds(0, last)],
                    gtab_s.at[pl.ds((NS - 1) * zrows, last)])


def _layer1_sc(h1, src3, dst3, acc_rows):
  k_chunks = src3.shape[1]
  n_rows, hid = h1.shape
  zrows = acc_rows // NS
  zpad = ((zrows + 15) // 16) * 16
  offs = _row_offs(hid)

  @functools.partial(
      pl.kernel,
      out_type=(
          jax.ShapeDtypeStruct((NC, acc_rows, hid), jnp.float32),
          jax.ShapeDtypeStruct((acc_rows,), jnp.float32),
      ),
      mesh=_sc_mesh(),
      compiler_params=pltpu.CompilerParams(use_tc_tiling_on_sc=False, needs_layout_passes=False),
      scratch_types=[
          pltpu.VMEM((k_chunks, CHUNK), jnp.int32),
          pltpu.VMEM((k_chunks, CHUNK), jnp.int32),
          pltpu.VMEM((2 * k_chunks, CHUNK), jnp.int32),
          pltpu.VMEM((CHUNK,), jnp.float32),
          pltpu.VMEM((zpad,), jnp.float32),
          pltpu.VMEM((NBUF * CHUNK, hid), jnp.float32),
          pltpu.VMEM_SHARED((acc_rows,), jnp.float32),
          pltpu.VMEM_SHARED((acc_rows, hid), jnp.float32),
          pltpu.VMEM_SHARED((acc_rows, hid), jnp.float32),
      ] + [pltpu.SemaphoreType.DMA] * (2 * NBUF + DSEM),
  )
  def run(h1_hbm, src_hbm, dst_hbm, out_hbm, dis_hbm,
          src_v, dst_v, ddst_v, ones_v, dis_v, rows_v,
          deg_s, gtab_s, acc_s, *sems):
    gsem = sems[:NBUF]
    ssem = sems[NBUF:2 * NBUF]
    dsem = sems[2 * NBUF:]
    c = lax.axis_index("c")
    s = lax.axis_index("s")
    w = c * NS + s
    pltpu.sync_copy(src_hbm.at[w], src_v)
    pltpu.sync_copy(dst_hbm.at[w], dst_v)
    pltpu.sync_copy(dst_hbm.at[2 * s], ddst_v.at[pl.ds(0, k_chunks)])
    pltpu.sync_copy(dst_hbm.at[2 * s + 1],
                    ddst_v.at[pl.ds(k_chunks, k_chunks)])

    @pl.loop(0, CHUNK // 16)
    def _fill1(i):
      ones_v[pl.ds(i * 16, 16)] = jnp.ones((16,), jnp.float32)

    @pl.loop(0, zpad // 16)
    def _fill0(i):
      dis_v[pl.ds(i * 16, 16)] = jnp.zeros((16,), jnp.float32)

    pltpu.sync_copy(dis_v.at[pl.ds(0, zrows)],
                    deg_s.at[pl.ds(s * zrows, zrows)])
    plsc.subcore_barrier()

    dk = 2 * k_chunks
    for b in range(DSEM):
      pltpu.async_copy(ones_v, deg_s.at[ddst_v.at[b]], dsem[b])

    @pl.loop(1, dk // DSEM)
    def _deg(g):
      for b in range(DSEM):
        k = g * DSEM + b
        pltpu.make_async_copy(ones_v, deg_s.at[ddst_v.at[k - DSEM]],
                              dsem[b]).wait()
        pltpu.async_copy(ones_v, deg_s.at[ddst_v.at[k]], dsem[b])

    for b in range(DSEM):
      pltpu.make_async_copy(ones_v, deg_s.at[ddst_v.at[dk - DSEM + b]],
                            dsem[b]).wait()
    plsc.subcore_barrier()

    pltpu.sync_copy(deg_s.at[pl.ds(s * zrows, zrows)],
                    dis_v.at[pl.ds(0, zrows)])

    @pl.loop(0, zpad // 16)
    def _dis(i):
      d = dis_v[pl.ds(i * 16, 16)] + 1.0
      dis_v[pl.ds(i * 16, 16)] = _qrsqrt(d)

    @pl.when(c == 0)
    def _dis_out():
      pltpu.sync_copy(dis_v.at[pl.ds(0, zrows)],
                      dis_hbm.at[pl.ds(s * zrows, zrows)])

    _stage_scaled(h1_hbm, dis_v, rows_v, gtab_s, s, zrows, hid, n_rows)

    _fill_zero_rows(rows_v, zrows, offs)
    pltpu.sync_copy(rows_v.at[pl.ds(0, zrows)],
                    acc_s.at[pl.ds(s * zrows, zrows)])
    plsc.subcore_barrier()

    _edge_ring(gtab_s, acc_s, src_v, dst_v, rows_v, gsem, ssem, k_chunks)

    plsc.subcore_barrier()
    pltpu.sync_copy(acc_s.at[pl.ds(s * zrows, zrows)],
                    rows_v.at[pl.ds(0, zrows)])
    pltpu.sync_copy(rows_v.at[pl.ds(0, zrows)],
                    out_hbm.at[c, pl.ds(s * zrows, zrows)])

  return run(h1, src3, dst3)


def _layer2_sc(h2, dis, src3, dst3, acc_rows):
  k_chunks = src3.shape[1]
  n_rows, feat = h2.shape
  zrows = acc_rows // NS
  zpad = ((zrows + 15) // 16) * 16
  offs = _row_offs(feat)

  @functools.partial(
      pl.kernel,
      out_type=jax.ShapeDtypeStruct((NC, acc_rows, feat), jnp.float32),
      mesh=_sc_mesh(),
      compiler_params=pltpu.CompilerParams(use_tc_tiling_on_sc=False, needs_layout_passes=False),
      scratch_types=[
          pltpu.VMEM((k_chunks, CHUNK), jnp.int32),
          pltpu.VMEM((k_chunks, CHUNK), jnp.int32),
          pltpu.VMEM((zpad,), jnp.float32),
          pltpu.VMEM((NBUF * CHUNK, feat), jnp.float32),
          pltpu.VMEM_SHARED((acc_rows, feat), jnp.float32),
          pltpu.VMEM_SHARED((acc_rows, feat), jnp.float32),
      ] + [pltpu.SemaphoreType.DMA] * (2 * NBUF),
  )
  def run(h2_hbm, dis_hbm, src_hbm, dst_hbm, out_hbm,
          src_v, dst_v, dis_v, rows_v, gtab_s, acc_s, *sems):
    gsem = sems[:NBUF]
    ssem = sems[NBUF:]
    c = lax.axis_index("c")
    s = lax.axis_index("s")
    w = c * NS + s
    pltpu.sync_copy(src_hbm.at[w], src_v)
    pltpu.sync_copy(dst_hbm.at[w], dst_v)
    pltpu.sync_copy(dis_hbm.at[pl.ds(s * zrows, zrows)],
                    dis_v.at[pl.ds(0, zrows)])

    _stage_scaled(h2_hbm, dis_v, rows_v, gtab_s, s, zrows, feat, n_rows)

    _fill_zero_rows(rows_v, zrows, offs)
    pltpu.sync_copy(rows_v.at[pl.ds(0, zrows)],
                    acc_s.at[pl.ds(s * zrows, zrows)])
    plsc.subcore_barrier()

    _edge_ring(gtab_s, acc_s, src_v, dst_v, rows_v, gsem, ssem, k_chunks)

    plsc.subcore_barrier()
    pltpu.sync_copy(acc_s.at[pl.ds(s * zrows, zrows)],
                    rows_v.at[pl.ds(0, zrows)])
    pltpu.sync_copy(rows_v.at[pl.ds(0, zrows)],
                    out_hbm.at[c, pl.ds(s * zrows, zrows)])

  return run(h2, dis, src3, dst3)


def _tc_matmul(x, w1, bm):
  n, d_in = x.shape
  hid = w1.shape[1]

  def body(x_ref, w1_ref, h1_ref):
    h1_ref[...] = jnp.dot(x_ref[...], w1_ref[...],
                          preferred_element_type=jnp.float32)

  return pl.pallas_call(
      body,
      grid=(n // bm,),
      in_specs=[
          pl.BlockSpec((bm, d_in), lambda i: (i, 0)),
          pl.BlockSpec((d_in, hid), lambda i: (0, 0)),
      ],
      out_specs=pl.BlockSpec((bm, hid), lambda i: (i, 0)),
      out_shape=jax.ShapeDtypeStruct((n, hid), jnp.float32),
  )(x, w1)


def _tc_mid(acc_p, h1, dis, b1, w2, bm):
  n, hid = h1.shape
  acc_rows = dis.shape[0]
  ncls = w2.shape[1]

  def body(acc_ref, h1_ref, dis_ref, b1_ref, w2_ref, h2_ref):
    dis_c = dis_ref[...]
    a = dis_c * (acc_ref[0] + acc_ref[1]) + dis_c * dis_c * h1_ref[...]
    z = jnp.maximum(a + b1_ref[...], 0.0)
    h2_ref[...] = jnp.dot(z, w2_ref[...],
                          preferred_element_type=jnp.float32)

  return pl.pallas_call(
      body,
      grid=(n // bm,),
      in_specs=[
          pl.BlockSpec((NC, bm, hid), lambda i: (0, i, 0)),
          pl.BlockSpec((bm, hid), lambda i: (i, 0)),
          pl.BlockSpec((bm, 1), lambda i: (i, 0)),
          pl.BlockSpec((1, hid), lambda i: (0, 0)),
          pl.BlockSpec((hid, ncls), lambda i: (0, 0)),
      ],
      out_specs=pl.BlockSpec((bm, ncls), lambda i: (i, 0)),
      out_shape=jax.ShapeDtypeStruct((n, ncls), jnp.float32),
  )(acc_p, h1, dis.reshape(acc_rows, 1), b1, w2)


def _tc_last(acc_p, h2, dis, b2, bm):
  n, ncls = h2.shape
  acc_rows = dis.shape[0]

  def body(acc_ref, h2_ref, dis_ref, b2_ref, out_ref):
    dis_c = dis_ref[...]
    out_ref[...] = (dis_c * (acc_ref[0] + acc_ref[1])
                    + dis_c * dis_c * h2_ref[...] + b2_ref[...])

  return pl.pallas_call(
      body,
      grid=(n // bm,),
      in_specs=[
          pl.BlockSpec((NC, bm, ncls), lambda i: (0, i, 0)),
          pl.BlockSpec((bm, ncls), lambda i: (i, 0)),
          pl.BlockSpec((bm, 1), lambda i: (i, 0)),
          pl.BlockSpec((1, ncls), lambda i: (0, 0)),
      ],
      out_specs=pl.BlockSpec((bm, ncls), lambda i: (i, 0)),
      out_shape=jax.ShapeDtypeStruct((n, ncls), jnp.float32),
  )(acc_p, h2, dis.reshape(acc_rows, 1), b2)


def kernel(x, edge_index, W1, b1, W2, b2):
  n, _ = x.shape
  hid = W1.shape[1]
  ncls = W2.shape[1]
  e = edge_index.shape[1]

  acc_rows = ((n + 1 + 127) // 128) * 128
  bm = 2000

  ew = NW * CHUNK
  k_chunks = ((e + ew - 1) // ew + NBUF - 1) // NBUF * NBUF
  e_pad = k_chunks * ew
  src = edge_index[0]
  dst = edge_index[1]
  pad = e_pad - e
  src3 = jnp.concatenate(
      [src, jnp.zeros((pad,), jnp.int32)]).reshape(NW, k_chunks, CHUNK)
  dst3 = jnp.concatenate(
      [dst, jnp.full((pad,), n, jnp.int32)]).reshape(NW, k_chunks, CHUNK)

  h1 = _tc_matmul(x, W1, bm)
  acc1, dis = _layer1_sc(h1, src3, dst3, acc_rows)
  h2 = _tc_mid(acc1, h1, dis, b1.reshape(1, hid), W2, bm)
  acc2 = _layer2_sc(h2, dis, src3, dst3, acc_rows)
  return _tc_last(acc2, h2, dis, b2.reshape(1, ncls), bm)


# speedup vs baseline: 1.7853x; 1.0022x over previous
import functools

import jax
import jax.numpy as jnp
from jax import lax
from jax.experimental import pallas as pl
from jax.experimental.pallas import tpu as pltpu
from jax.experimental.pallas import tpu_sc as plsc

NC = 2
NS = 16
NW = NC * NS
CHUNK = 128
NBUF = 4
FIRE = 2
DSEM = 4


def _sc_mesh():
  return plsc.VectorSubcoreMesh(
      core_axis_name="c", subcore_axis_name="s", num_cores=NC,
      num_subcores=NS)


def _qrsqrt(d):
  i = plsc.bitcast(d, jnp.int32)
  i = jnp.int32(0x5F3759DF) - lax.shift_right_logical(i, 1)
  y = plsc.bitcast(i, jnp.float32)
  for _ in range(3):
    y = y * (1.5 - 0.5 * d * y * y)
  return y


def _row_offs(feat):
  offs = list(range(0, feat - 15, 16))
  if feat % 16:
    offs.append(feat - 16)
  return offs


def _fill_zero_rows(rows_v, nrows, offs):
  @pl.loop(0, nrows)
  def _fz(r):
    for o in offs:
      rows_v[r, pl.ds(o, 16)] = jnp.zeros((16,), jnp.float32)


def _edge_ring(gtab_s, acc_s, src_v, dst_v, rows_v, gsem, ssem, k_chunks):
  def buf(b):
    return rows_v.at[pl.ds(b * CHUNK, CHUNK)]

  def wait_gather(k, b):
    pltpu.make_async_copy(gtab_s.at[src_v.at[k]], buf(b), gsem[b]).wait()

  def wait_scatter(k, b):
    pltpu.make_async_copy(buf(b), acc_s.at[dst_v.at[k]], ssem[b]).wait()

  for b in range(FIRE):
    pltpu.async_copy(gtab_s.at[src_v.at[b]], buf(b), gsem[b])

  for b in range(NBUF):
    k = b
    wait_gather(k, b)
    pltpu.async_copy(buf(b), acc_s.at[dst_v.at[k]], ssem[b], add=True)
    nb = (k + FIRE) % NBUF
    if k + FIRE >= NBUF:
      wait_scatter(k + FIRE - NBUF, nb)
    pltpu.async_copy(gtab_s.at[src_v.at[k + FIRE]], buf(nb), gsem[nb])

  @pl.loop(1, k_chunks // NBUF)
  def _edges(g):
    for b in range(NBUF):
      k = g * NBUF + b
      wait_gather(k, b)
      pltpu.async_copy(buf(b), acc_s.at[dst_v.at[k]], ssem[b], add=True)
      nxt = k + FIRE
      nb = (b + FIRE) % NBUF

      @pl.when(nxt < k_chunks)
      def _refire():
        wait_scatter(nxt - NBUF, nb)
        pltpu.async_copy(gtab_s.at[src_v.at[nxt]], buf(nb), gsem[nb])

  for b in range(NBUF):
    k = k_chunks - NBUF + b
    wait_scatter(k, k % NBUF)


def _stage_scaled(h_hbm, dis_v, rows_v, gtab_s, s, zrows, feat, n_rows):
  last = n_rows - (NS - 1) * zrows
  offs = _row_offs(feat)

  def scale(nr):
    @pl.loop(0, (nr + 15) // 16)
    def _sc(i):
      dv = dis_v[pl.ds(i * 16, 16)]
      for j in range(16):
        r = i * 16 + j
        d = lax.broadcast_in_dim(dv[j], (16,), ())
        for o in offs:
          rows_v[r, pl.ds(o, 16)] = rows_v[r, pl.ds(o, 16)] * d

  @pl.when(s < NS - 1)
  def _full():
    pltpu.sync_copy(h_hbm.at[pl.ds(s * zrows, zrows)],
                    rows_v.at[pl.ds(0, zrows)])
    scale(zrows)
    pltpu.sync_copy(rows_v.at[pl.ds(0, zrows)],
                    gtab_s.at[pl.ds(s * zrows, zrows)])

  @pl.when(s == NS - 1)
  def _last():
    pltpu.sync_copy(h_hbm.at[pl.ds((NS - 1) * zrows, last)],
                    rows_v.at[pl.ds(0, last)])
    scale(last)
    pltpu.sync_copy(rows_v.at[pl.ds(0, last)],
                    gtab_s.at[pl.ds((NS - 1) * zrows, last)])


def _layer1_sc(h1, src3, dst3, acc_rows):
  k_chunks = src3.shape[1]
  n_rows, hid = h1.shape
  zrows = acc_rows // NS
  zpad = ((zrows + 15) // 16) * 16
  offs = _row_offs(hid)

  @functools.partial(
      pl.kernel,
      out_type=(
          jax.ShapeDtypeStruct((NC, acc_rows, hid), jnp.float32),
          jax.ShapeDtypeStruct((acc_rows,), jnp.float32),
      ),
      mesh=_sc_mesh(),
      compiler_params=pltpu.CompilerParams(use_tc_tiling_on_sc=False, needs_layout_passes=False),
      scratch_types=[
          pltpu.VMEM((k_chunks, CHUNK), jnp.int32),
          pltpu.VMEM((k_chunks, CHUNK), jnp.int32),
          pltpu.VMEM((2 * k_chunks, CHUNK), jnp.int32),
          pltpu.VMEM((CHUNK,), jnp.float32),
          pltpu.VMEM((zpad,), jnp.float32),
          pltpu.VMEM((NBUF * CHUNK, hid), jnp.float32),
          pltpu.VMEM_SHARED((acc_rows,), jnp.float32),
          pltpu.VMEM_SHARED((acc_rows, hid), jnp.float32),
          pltpu.VMEM_SHARED((acc_rows, hid), jnp.float32),
      ] + [pltpu.SemaphoreType.DMA] * (2 * NBUF + DSEM),
  )
  def run(h1_hbm, src_hbm, dst_hbm, out_hbm, dis_hbm,
          src_v, dst_v, ddst_v, ones_v, dis_v, rows_v,
          deg_s, gtab_s, acc_s, *sems):
    gsem = sems[:NBUF]
    ssem = sems[NBUF:2 * NBUF]
    dsem = sems[2 * NBUF:]
    c = lax.axis_index("c")
    s = lax.axis_index("s")
    w = c * NS + s
    pltpu.sync_copy(src_hbm.at[w], src_v)
    pltpu.sync_copy(dst_hbm.at[w], dst_v)
    pltpu.sync_copy(dst_hbm.at[2 * s], ddst_v.at[pl.ds(0, k_chunks)])
    pltpu.sync_copy(dst_hbm.at[2 * s + 1],
                    ddst_v.at[pl.ds(k_chunks, k_chunks)])

    @pl.loop(0, CHUNK // 16)
    def _fill1(i):
      ones_v[pl.ds(i * 16, 16)] = jnp.ones((16,), jnp.float32)

    @pl.loop(0, zpad // 16)
    def _fill0(i):
      dis_v[pl.ds(i * 16, 16)] = jnp.zeros((16,), jnp.float32)

    pltpu.sync_copy(dis_v.at[pl.ds(0, zrows)],
                    deg_s.at[pl.ds(s * zrows, zrows)])
    plsc.subcore_barrier()

    dk = 2 * k_chunks
    for b in range(DSEM):
      pltpu.async_copy(ones_v, deg_s.at[ddst_v.at[b]], dsem[b])

    @pl.loop(1, dk // DSEM)
    def _deg(g):
      for b in range(DSEM):
        k = g * DSEM + b
        pltpu.make_async_copy(ones_v, deg_s.at[ddst_v.at[k - DSEM]],
                              dsem[b]).wait()
        pltpu.async_copy(ones_v, deg_s.at[ddst_v.at[k]], dsem[b])

    for b in range(DSEM):
      pltpu.make_async_copy(ones_v, deg_s.at[ddst_v.at[dk - DSEM + b]],
                            dsem[b]).wait()
    plsc.subcore_barrier()

    pltpu.sync_copy(deg_s.at[pl.ds(s * zrows, zrows)],
                    dis_v.at[pl.ds(0, zrows)])

    @pl.loop(0, zpad // 16)
    def _dis(i):
      d = dis_v[pl.ds(i * 16, 16)] + 1.0
      dis_v[pl.ds(i * 16, 16)] = _qrsqrt(d)

    @pl.when(c == 0)
    def _dis_out():
      pltpu.sync_copy(dis_v.at[pl.ds(0, zrows)],
                      dis_hbm.at[pl.ds(s * zrows, zrows)])

    _stage_scaled(h1_hbm, dis_v, rows_v, gtab_s, s, zrows, hid, n_rows)

    _fill_zero_rows(rows_v, zrows, offs)
    pltpu.sync_copy(rows_v.at[pl.ds(0, zrows)],
                    acc_s.at[pl.ds(s * zrows, zrows)])
    plsc.subcore_barrier()

    _edge_ring(gtab_s, acc_s, src_v, dst_v, rows_v, gsem, ssem, k_chunks)

    plsc.subcore_barrier()
    pltpu.sync_copy(acc_s.at[pl.ds(s * zrows, zrows)],
                    rows_v.at[pl.ds(0, zrows)])
    pltpu.sync_copy(rows_v.at[pl.ds(0, zrows)],
                    out_hbm.at[c, pl.ds(s * zrows, zrows)])

  return run(h1, src3, dst3)


def _layer2_sc(h2, dis, src3, dst3, acc_rows):
  k_chunks = src3.shape[1]
  n_rows, feat = h2.shape
  zrows = acc_rows // NS
  zpad = ((zrows + 15) // 16) * 16
  offs = _row_offs(feat)

  @functools.partial(
      pl.kernel,
      out_type=jax.ShapeDtypeStruct((NC, acc_rows, feat), jnp.float32),
      mesh=_sc_mesh(),
      compiler_params=pltpu.CompilerParams(use_tc_tiling_on_sc=False, needs_layout_passes=False),
      scratch_types=[
          pltpu.VMEM((k_chunks, CHUNK), jnp.int32),
          pltpu.VMEM((k_chunks, CHUNK), jnp.int32),
          pltpu.VMEM((zpad,), jnp.float32),
          pltpu.VMEM((NBUF * CHUNK, feat), jnp.float32),
          pltpu.VMEM_SHARED((acc_rows, feat), jnp.float32),
          pltpu.VMEM_SHARED((acc_rows, feat), jnp.float32),
      ] + [pltpu.SemaphoreType.DMA] * (2 * NBUF),
  )
  def run(h2_hbm, dis_hbm, src_hbm, dst_hbm, out_hbm,
          src_v, dst_v, dis_v, rows_v, gtab_s, acc_s, *sems):
    gsem = sems[:NBUF]
    ssem = sems[NBUF:]
    c = lax.axis_index("c")
    s = lax.axis_index("s")
    w = c * NS + s
    pltpu.sync_copy(src_hbm.at[w], src_v)
    pltpu.sync_copy(dst_hbm.at[w], dst_v)
    pltpu.sync_copy(dis_hbm.at[pl.ds(s * zrows, zrows)],
                    dis_v.at[pl.ds(0, zrows)])

    _stage_scaled(h2_hbm, dis_v, rows_v, gtab_s, s, zrows, feat, n_rows)

    _fill_zero_rows(rows_v, zrows, offs)
    pltpu.sync_copy(rows_v.at[pl.ds(0, zrows)],
                    acc_s.at[pl.ds(s * zrows, zrows)])
    plsc.subcore_barrier()

    _edge_ring(gtab_s, acc_s, src_v, dst_v, rows_v, gsem, ssem, k_chunks)

    plsc.subcore_barrier()
    pltpu.sync_copy(acc_s.at[pl.ds(s * zrows, zrows)],
                    rows_v.at[pl.ds(0, zrows)])
    pltpu.sync_copy(rows_v.at[pl.ds(0, zrows)],
                    out_hbm.at[c, pl.ds(s * zrows, zrows)])

  return run(h2, dis, src3, dst3)


def _tc_matmul(x, w1, bm):
  n, d_in = x.shape
  hid = w1.shape[1]

  def body(x_ref, w1_ref, h1_ref):
    h1_ref[...] = jnp.dot(x_ref[...], w1_ref[...],
                          preferred_element_type=jnp.float32)

  return pl.pallas_call(
      body,
      grid=(n // bm,),
      in_specs=[
          pl.BlockSpec((bm, d_in), lambda i: (i, 0)),
          pl.BlockSpec((d_in, hid), lambda i: (0, 0)),
      ],
      out_specs=pl.BlockSpec((bm, hid), lambda i: (i, 0)),
      out_shape=jax.ShapeDtypeStruct((n, hid), jnp.float32),
  )(x, w1)


def _tc_mid(acc_p, h1, dis, b1, w2, bm):
  n, hid = h1.shape
  acc_rows = dis.shape[0]
  ncls = w2.shape[1]

  def body(acc_ref, h1_ref, dis_ref, b1_ref, w2_ref, h2_ref):
    dis_c = dis_ref[...]
    a = dis_c * (acc_ref[0] + acc_ref[1]) + dis_c * dis_c * h1_ref[...]
    z = jnp.maximum(a + b1_ref[...], 0.0)
    h2_ref[...] = jnp.dot(z, w2_ref[...],
                          preferred_element_type=jnp.float32)

  return pl.pallas_call(
      body,
      grid=(n // bm,),
      in_specs=[
          pl.BlockSpec((NC, bm, hid), lambda i: (0, i, 0)),
          pl.BlockSpec((bm, hid), lambda i: (i, 0)),
          pl.BlockSpec((bm, 1), lambda i: (i, 0)),
          pl.BlockSpec((1, hid), lambda i: (0, 0)),
          pl.BlockSpec((hid, ncls), lambda i: (0, 0)),
      ],
      out_specs=pl.BlockSpec((bm, ncls), lambda i: (i, 0)),
      out_shape=jax.ShapeDtypeStruct((n, ncls), jnp.float32),
  )(acc_p, h1, dis.reshape(acc_rows, 1), b1, w2)


def _tc_last(acc_p, h2, dis, b2, bm):
  n, ncls = h2.shape
  acc_rows = dis.shape[0]

  def body(acc_ref, h2_ref, dis_ref, b2_ref, out_ref):
    dis_c = dis_ref[...]
    out_ref[...] = (dis_c * (acc_ref[0] + acc_ref[1])
                    + dis_c * dis_c * h2_ref[...] + b2_ref[...])

  return pl.pallas_call(
      body,
      grid=(n // bm,),
      in_specs=[
          pl.BlockSpec((NC, bm, ncls), lambda i: (0, i, 0)),
          pl.BlockSpec((bm, ncls), lambda i: (i, 0)),
          pl.BlockSpec((bm, 1), lambda i: (i, 0)),
          pl.BlockSpec((1, ncls), lambda i: (0, 0)),
      ],
      out_specs=pl.BlockSpec((bm, ncls), lambda i: (i, 0)),
      out_shape=jax.ShapeDtypeStruct((n, ncls), jnp.float32),
  )(acc_p, h2, dis.reshape(acc_rows, 1), b2)


def kernel(x, edge_index, W1, b1, W2, b2):
  n, _ = x.shape
  hid = W1.shape[1]
  ncls = W2.shape[1]
  e = edge_index.shape[1]

  acc_rows = ((n + 1 + 127) // 128) * 128
  bm = 2000

  ew = NW * CHUNK
  k_chunks = ((e + ew - 1) // ew + NBUF - 1) // NBUF * NBUF
  e_pad = k_chunks * ew
  src = edge_index[0]
  dst = edge_index[1]
  pad = e_pad - e
  src3 = jnp.concatenate(
      [src, jnp.zeros((pad,), jnp.int32)]).reshape(NW, k_chunks, CHUNK)
  dst3 = jnp.concatenate(
      [dst, jnp.full((pad,), n, jnp.int32)]).reshape(NW, k_chunks, CHUNK)

  h1 = _tc_matmul(x, W1, bm)
  acc1, dis = _layer1_sc(h1, src3, dst3, acc_rows)
  h2 = _tc_mid(acc1, h1, dis, b1.reshape(1, hid), W2, bm)
  acc2 = _layer2_sc(h2, dis, src3, dst3, acc_rows)
  return _tc_last(acc2, h2, dis, b2.reshape(1, ncls), bm)
